# baseline (speedup 1.0000x reference)
import jax
import jax.numpy as jnp
from jax import lax
from jax.experimental import pallas as pl
from jax.experimental.pallas import tpu as pltpu


def kernel(x, row_W, col_W, pose_W, p):
    b, _, h, w = x.shape
    d = row_W.shape[1]
    hw = h * w
    flat = 2 * d * hw
    assert flat % 128 == 0
    rows = flat // 128

    def body_a(p_ref, row_ref, col_ref, pose_ref, pat_ref, pemb_ref):
        u = lax.broadcasted_iota(jnp.int32, (h, hw), 1)
        j = lax.broadcasted_iota(jnp.int32, (h, hw), 0)
        s_col = (u % w == j).astype(jnp.float32)
        s_row = (u // w == j).astype(jnp.float32)
        dn = (((0,), (0,)), ((), ()))
        top = lax.dot_general(col_ref[1:w + 1, :], s_col, dn,
                              preferred_element_type=jnp.float32)
        bot = lax.dot_general(row_ref[1:h + 1, :], s_row, dn,
                              preferred_element_type=jnp.float32)
        pat_ref[...] = jnp.concatenate([top, bot], axis=0)
        half = jnp.broadcast_to(pose_ref[p_ref[0], :][None, :], (b, d))
        pemb_ref[...] = jnp.concatenate([half, half], axis=1)

    grid_spec = pltpu.PrefetchScalarGridSpec(
        num_scalar_prefetch=1,
        grid=(1,),
        in_specs=[
            pl.BlockSpec(row_W.shape, lambda i, p_: (0, 0)),
            pl.BlockSpec(col_W.shape, lambda i, p_: (0, 0)),
            pl.BlockSpec(pose_W.shape, lambda i, p_: (0, 0)),
        ],
        out_specs=[
            pl.BlockSpec((2 * d, hw), lambda i, p_: (0, 0)),
            pl.BlockSpec((b, 2 * d), lambda i, p_: (0, 0)),
        ],
    )
    pat, p_emb = pl.pallas_call(
        body_a,
        grid_spec=grid_spec,
        out_shape=[
            jax.ShapeDtypeStruct((2 * d, hw), jnp.float32),
            jax.ShapeDtypeStruct((b, 2 * d), jnp.float32),
        ],
    )(jnp.reshape(p, (1,)).astype(jnp.int32), row_W, col_W, pose_W)

    v = jnp.reshape(pat, (rows, 128))

    def body_b(v_ref, memb_ref):
        memb_ref[0] = v_ref[...]

    m_flat = pl.pallas_call(
        body_b,
        grid=(b,),
        in_specs=[pl.BlockSpec((rows, 128), lambda bi: (0, 0))],
        out_specs=pl.BlockSpec((1, rows, 128), lambda bi: (bi, 0, 0)),
        out_shape=jax.ShapeDtypeStruct((b, rows, 128), jnp.float32),
    )(v)

    return (p_emb, jnp.reshape(m_flat, (b, 2 * d, h, w)))


# speedup vs baseline: 17.7959x; 17.7959x over previous
import jax
import jax.numpy as jnp
from jax.experimental import pallas as pl
from jax.experimental.pallas import tpu as pltpu


def kernel(x, row_W, col_W, pose_W, p):
    b, _, h, w = x.shape
    d = row_W.shape[1]

    def body(p_ref, row_ref, col_ref, pose_ref, mt_ref, pemb_ref):
        bi = pl.program_id(0)
        col_s = col_ref[1:w + 1, :]
        row_s = row_ref[1:h + 1, :]
        bc_col = jnp.broadcast_to(col_s[None, :, :], (h, w, d))
        bc_row = jnp.broadcast_to(row_s[:, None, :], (h, w, d))
        mt_ref[0] = jnp.concatenate([bc_col, bc_row], axis=-1)

        @pl.when(bi == 0)
        def _():
            half = jnp.broadcast_to(pose_ref[p_ref[0], :][None, :], (b, d))
            pemb_ref[...] = jnp.concatenate([half, half], axis=1)

    grid_spec = pltpu.PrefetchScalarGridSpec(
        num_scalar_prefetch=1,
        grid=(b,),
        in_specs=[
            pl.BlockSpec(row_W.shape, lambda bi, p_: (0, 0)),
            pl.BlockSpec(col_W.shape, lambda bi, p_: (0, 0)),
            pl.BlockSpec(pose_W.shape, lambda bi, p_: (0, 0)),
        ],
        out_specs=[
            pl.BlockSpec((1, h, w, 2 * d), lambda bi, p_: (bi, 0, 0, 0)),
            pl.BlockSpec((b, 2 * d), lambda bi, p_: (0, 0)),
        ],
    )
    m_t, p_emb = pl.pallas_call(
        body,
        grid_spec=grid_spec,
        out_shape=[
            jax.ShapeDtypeStruct((b, h, w, 2 * d), jnp.float32),
            jax.ShapeDtypeStruct((b, 2 * d), jnp.float32),
        ],
    )(jnp.reshape(p, (1,)).astype(jnp.int32), row_W, col_W, pose_W)

    return (p_emb, jnp.transpose(m_t, (0, 3, 1, 2)))


# speedup vs baseline: 23.9622x; 1.3465x over previous
import jax
import jax.numpy as jnp
from jax.experimental import pallas as pl
from jax.experimental.pallas import tpu as pltpu


def kernel(x, row_W, col_W, pose_W, p):
    b, _, h, w = x.shape
    d = row_W.shape[1]

    def body(p_ref, row_ref, col_ref, pose_ref, mt_ref, pemb_ref,
             scratch, sem):
        col_s = col_ref[1:w + 1, :]
        row_s = row_ref[1:h + 1, :]
        bc_col = jnp.broadcast_to(col_s[None, :, :], (h, w, d))
        bc_row = jnp.broadcast_to(row_s[:, None, :], (h, w, d))
        scratch[...] = jnp.concatenate([bc_col, bc_row], axis=-1)
        half = jnp.broadcast_to(pose_ref[p_ref[0], :][None, :], (b, d))
        pemb_ref[...] = jnp.concatenate([half, half], axis=1)
        copies = [pltpu.make_async_copy(scratch, mt_ref.at[i], sem)
                  for i in range(b)]
        for c in copies:
            c.start()
        for c in copies:
            c.wait()

    grid_spec = pltpu.PrefetchScalarGridSpec(
        num_scalar_prefetch=1,
        grid=(1,),
        in_specs=[
            pl.BlockSpec(row_W.shape, lambda i, p_: (0, 0)),
            pl.BlockSpec(col_W.shape, lambda i, p_: (0, 0)),
            pl.BlockSpec(pose_W.shape, lambda i, p_: (0, 0)),
        ],
        out_specs=[
            pl.BlockSpec(memory_space=pl.ANY),
            pl.BlockSpec((b, 2 * d), lambda i, p_: (0, 0)),
        ],
        scratch_shapes=[
            pltpu.VMEM((h, w, 2 * d), jnp.float32),
            pltpu.SemaphoreType.DMA,
        ],
    )
    m_t, p_emb = pl.pallas_call(
        body,
        grid_spec=grid_spec,
        out_shape=[
            jax.ShapeDtypeStruct((b, h, w, 2 * d), jnp.float32),
            jax.ShapeDtypeStruct((b, 2 * d), jnp.float32),
        ],
    )(jnp.reshape(p, (1,)).astype(jnp.int32), row_W, col_W, pose_W)

    return (p_emb, jnp.transpose(m_t, (0, 3, 1, 2)))
